# baseline (speedup 1.0000x reference)
import jax
import jax.numpy as jnp
import numpy as np
from jax.experimental import pallas as pl
from jax.experimental.pallas import tpu as pltpu

B = 16
M = 2048
D = 256
H = 8
A = 64
O = 64
HO = H * O
TM = 512
NT = M // TM
R = 4
SH = R * H
OUTW = HO + 128


def _agg_kernel(xf_ref, wq_ref, bq_ref, wk_ref, bk_ref, wv_ref, bv_ref,
                seg_ref, exp_ref, out_ref, sc_ref):
    ens = [jnp.sum(xf_ref[r, M:, :]) for r in range(R)]

    for t in range(NT):
        s = None
        for r in range(R):
            xt = xf_ref[r, t * TM:(t + 1) * TM, :].astype(jnp.bfloat16)
            q = jnp.dot(xt, wq_ref[...], preferred_element_type=jnp.float32) + bq_ref[...]
            k = jnp.dot(xt, wk_ref[...], preferred_element_type=jnp.float32) + bk_ref[...]
            d = jnp.dot((q * k).astype(jnp.bfloat16), seg_ref[r],
                        preferred_element_type=jnp.float32)
            s = d if s is None else s + d
        sc_ref[t * TM:(t + 1) * TM, :] = s

    en_vec = jnp.concatenate(
        [jnp.full((1, H), ens[r], jnp.float32) for r in range(R)], axis=1)
    rows = jax.lax.broadcasted_iota(jnp.int32, (M, SH), 0)
    pm = jnp.where(rows.astype(jnp.float32) < en_vec, 1.0, 0.0)
    z = sc_ref[...] * pm
    zmax = jnp.max(jnp.maximum(z, 0.0), axis=0, keepdims=True)
    ez = jnp.exp(z - zmax) * pm
    denom = jnp.sum(ez, axis=0, keepdims=True) + 1.0
    wts = ez / denom

    for r in range(R):
        w_r = wts[:, H * r:H * (r + 1)]
        xbar = jax.lax.dot_general(w_r, xf_ref[r, :M, :],
                                   (((0,), (0,)), ((), ())),
                                   preferred_element_type=jnp.float32)
        hsum = jnp.sum(w_r, axis=0, keepdims=True)
        vout = jnp.dot(xbar, wv_ref[...], preferred_element_type=jnp.float32)
        acc = (jnp.sum(vout * exp_ref[...], axis=0, keepdims=True)
               + jnp.dot(hsum, exp_ref[...], preferred_element_type=jnp.float32)
               * bv_ref[...])
        frac = jnp.full((1, OUTW - HO), ens[r] * (1.0 / M), jnp.float32)
        out_ref[r] = jnp.concatenate([acc, frac], axis=1)


NSPLIT = 2
BS = B // NSPLIT


def kernel(x, Ws, bs, Wq, bq, Wk, bk, Wv, bv):
    xhalves = [x[i * BS:(i + 1) * BS].reshape(BS, M + 8, D)
               for i in range(NSPLIT)]

    wq_f = (Ws @ Wq).astype(jnp.bfloat16)
    bq_f = (bs @ Wq + bq).reshape(1, HO)
    wk_f = (Ws @ Wk).astype(jnp.bfloat16)
    bk_f = (bs @ Wk + bk).reshape(1, HO)
    wv_f = Ws @ Wv
    bv_f = (bs @ Wv + bv).reshape(1, HO)

    seg_np = np.zeros((R, HO, SH), np.float32)
    for r in range(R):
        for h in range(H):
            seg_np[r, h * A:(h + 1) * A, H * r + h] = 1.0 / np.sqrt(A)
    seg = jnp.asarray(seg_np, jnp.bfloat16)
    exp = jnp.asarray(np.kron(np.eye(H), np.ones((1, O))), jnp.float32)

    const = lambda b: (0, 0)
    call = pl.pallas_call(
        _agg_kernel,
        grid=(BS // R,),
        in_specs=[
            pl.BlockSpec((R, M + 8, D), lambda b: (b, 0, 0)),
            pl.BlockSpec((D, HO), const),
            pl.BlockSpec((1, HO), const),
            pl.BlockSpec((D, HO), const),
            pl.BlockSpec((1, HO), const),
            pl.BlockSpec((D, HO), const),
            pl.BlockSpec((1, HO), const),
            pl.BlockSpec((R, HO, SH), lambda b: (0, 0, 0)),
            pl.BlockSpec((H, HO), const),
        ],
        out_specs=pl.BlockSpec((R, 1, OUTW), lambda b: (b, 0, 0)),
        out_shape=jax.ShapeDtypeStruct((BS, 1, OUTW), jnp.float32),
        scratch_shapes=[pltpu.VMEM((M, SH), jnp.float32)],
        compiler_params=pltpu.CompilerParams(
            dimension_semantics=("parallel",)),
    )
    outs = [call(xh, wq_f, bq_f, wk_f, bk_f, wv_f, bv_f, seg, exp)
            for xh in xhalves]
    out = jnp.concatenate(outs, axis=0)
    return out.reshape(B, OUTW)[:, : HO + 1]


# speedup vs baseline: 1.1091x; 1.1091x over previous
import jax
import jax.numpy as jnp
import numpy as np
from jax.experimental import pallas as pl
from jax.experimental.pallas import tpu as pltpu

B = 16
M = 2048
D = 256
H = 8
A = 64
O = 64
HO = H * O
TM = 512
NT = M // TM
R = 4
SH = R * H
OUTW = HO + 128


def _agg_kernel(xf_ref, xm_ref, wq_ref, bq_ref, wk_ref, bk_ref, wv_ref, bv_ref,
                seg_ref, exp_ref, out_ref, sc_ref):
    ens = [jnp.sum(xm_ref[r]) for r in range(R)]

    for t in range(NT):
        s = None
        for r in range(R):
            xt = xf_ref[r, t * TM:(t + 1) * TM, :]
            q = jnp.dot(xt, wq_ref[...], preferred_element_type=jnp.float32) + bq_ref[...]
            k = jnp.dot(xt, wk_ref[...], preferred_element_type=jnp.float32) + bk_ref[...]
            d = jnp.dot((q * k).astype(jnp.bfloat16), seg_ref[r],
                        preferred_element_type=jnp.float32)
            s = d if s is None else s + d
        sc_ref[t * TM:(t + 1) * TM, :] = s

    en_vec = jnp.concatenate(
        [jnp.full((1, H), ens[r], jnp.float32) for r in range(R)], axis=1)
    rows = jax.lax.broadcasted_iota(jnp.int32, (M, SH), 0)
    pm = jnp.where(rows.astype(jnp.float32) < en_vec, 1.0, 0.0)
    z = sc_ref[...] * pm
    zmax = jnp.max(jnp.maximum(z, 0.0), axis=0, keepdims=True)
    ez = jnp.exp(z - zmax) * pm
    denom = jnp.sum(ez, axis=0, keepdims=True) + 1.0
    wts = ez / denom

    for r in range(R):
        w_r = wts[:, H * r:H * (r + 1)]
        xbar = jax.lax.dot_general(w_r.astype(jnp.bfloat16), xf_ref[r],
                                   (((0,), (0,)), ((), ())),
                                   preferred_element_type=jnp.float32)
        hsum = jnp.sum(w_r, axis=0, keepdims=True)
        vout = jnp.dot(xbar, wv_ref[...], preferred_element_type=jnp.float32)
        acc = (jnp.sum(vout * exp_ref[...], axis=0, keepdims=True)
               + jnp.dot(hsum, exp_ref[...], preferred_element_type=jnp.float32)
               * bv_ref[...])
        frac = jnp.full((1, OUTW - HO), ens[r] * (1.0 / M), jnp.float32)
        out_ref[r] = jnp.concatenate([acc, frac], axis=1)


def kernel(x, Ws, bs, Wq, bq, Wk, bk, Wv, bv):
    xf = x[:, : M * D].reshape(B, M, D).astype(jnp.bfloat16)
    xm = x[:, M * D:].reshape(B, 1, M)

    wq_f = (Ws @ Wq).astype(jnp.bfloat16)
    bq_f = (bs @ Wq + bq).reshape(1, HO)
    wk_f = (Ws @ Wk).astype(jnp.bfloat16)
    bk_f = (bs @ Wk + bk).reshape(1, HO)
    wv_f = Ws @ Wv
    bv_f = (bs @ Wv + bv).reshape(1, HO)

    seg_np = np.zeros((R, HO, SH), np.float32)
    for r in range(R):
        for h in range(H):
            seg_np[r, h * A:(h + 1) * A, H * r + h] = 1.0 / np.sqrt(A)
    seg = jnp.asarray(seg_np, jnp.bfloat16)
    exp = jnp.asarray(np.kron(np.eye(H), np.ones((1, O))), jnp.float32)

    const = lambda b: (0, 0)
    call = pl.pallas_call(
        _agg_kernel,
        grid=(B // R,),
        in_specs=[
            pl.BlockSpec((R, M, D), lambda b: (b, 0, 0)),
            pl.BlockSpec((R, 1, M), lambda b: (b, 0, 0)),
            pl.BlockSpec((D, HO), const),
            pl.BlockSpec((1, HO), const),
            pl.BlockSpec((D, HO), const),
            pl.BlockSpec((1, HO), const),
            pl.BlockSpec((D, HO), const),
            pl.BlockSpec((1, HO), const),
            pl.BlockSpec((R, HO, SH), lambda b: (0, 0, 0)),
            pl.BlockSpec((H, HO), const),
        ],
        out_specs=pl.BlockSpec((R, 1, OUTW), lambda b: (b, 0, 0)),
        out_shape=jax.ShapeDtypeStruct((B, 1, OUTW), jnp.float32),
        scratch_shapes=[pltpu.VMEM((M, SH), jnp.float32)],
        compiler_params=pltpu.CompilerParams(
            dimension_semantics=("parallel",)),
    )
    out = call(xf, xm, wq_f, bq_f, wk_f, bk_f, wv_f, bv_f, seg, exp)
    return out.reshape(B, OUTW)[:, : HO + 1]


# speedup vs baseline: 1.1431x; 1.0307x over previous
import jax
import jax.numpy as jnp
import numpy as np
from jax.experimental import pallas as pl
from jax.experimental.pallas import tpu as pltpu

B = 16
M = 2048
D = 256
H = 8
A = 64
O = 64
HO = H * O
TM = 512
NT = M // TM
R = 4
SH = R * H
OUTW = HO + 128


def _agg_kernel(xf_ref, wqk_ref, bqk_ref, wv_ref, bv_ref,
                seg_ref, exp_ref, out_ref, sc_ref):
    ens = [jnp.sum(xf_ref[r, M:, :]) for r in range(R)]

    for t in range(NT):
        s = None
        for r in range(R):
            xt = xf_ref[r, t * TM:(t + 1) * TM, :].astype(jnp.bfloat16)
            qk2 = jnp.dot(xt, wqk_ref[...],
                          preferred_element_type=jnp.float32) + bqk_ref[...]
            q = qk2[:, :HO]
            k = qk2[:, HO:]
            d = jnp.dot((q * k).astype(jnp.bfloat16), seg_ref[r],
                        preferred_element_type=jnp.float32)
            s = d if s is None else s + d
        sc_ref[t * TM:(t + 1) * TM, :] = s

    en_vec = jnp.concatenate(
        [jnp.full((1, H), ens[r], jnp.float32) for r in range(R)], axis=1)
    rows = jax.lax.broadcasted_iota(jnp.int32, (M, SH), 0)
    pm = jnp.where(rows.astype(jnp.float32) < en_vec, 1.0, 0.0)
    z = sc_ref[...] * pm
    zmax = jnp.max(jnp.maximum(z, 0.0), axis=0, keepdims=True)
    ez = jnp.exp(z - zmax) * pm
    denom = jnp.sum(ez, axis=0, keepdims=True) + 1.0
    wts = ez / denom

    for r in range(R):
        w_r = wts[:, H * r:H * (r + 1)]
        xbar = jax.lax.dot_general(w_r, xf_ref[r, :M, :],
                                   (((0,), (0,)), ((), ())),
                                   preferred_element_type=jnp.float32)
        hsum = jnp.sum(w_r, axis=0, keepdims=True)
        vout = jnp.dot(xbar, wv_ref[...], preferred_element_type=jnp.float32)
        acc = (jnp.sum(vout * exp_ref[...], axis=0, keepdims=True)
               + jnp.dot(hsum, exp_ref[...], preferred_element_type=jnp.float32)
               * bv_ref[...])
        frac = jnp.full((1, OUTW - HO), ens[r] * (1.0 / M), jnp.float32)
        out_ref[r] = jnp.concatenate([acc, frac], axis=1)


def kernel(x, Ws, bs, Wq, bq, Wk, bk, Wv, bv):
    xall = x.reshape(B, M + 8, D)

    wqk_f = jnp.concatenate([Ws @ Wq, Ws @ Wk], axis=1).astype(jnp.bfloat16)
    bqk_f = jnp.concatenate([bs @ Wq + bq, bs @ Wk + bk]).reshape(1, 2 * HO)
    wv_f = Ws @ Wv
    bv_f = (bs @ Wv + bv).reshape(1, HO)

    seg_np = np.zeros((R, HO, SH), np.float32)
    for r in range(R):
        for h in range(H):
            seg_np[r, h * A:(h + 1) * A, H * r + h] = 1.0 / np.sqrt(A)
    seg = jnp.asarray(seg_np, jnp.bfloat16)
    exp = jnp.asarray(np.kron(np.eye(H), np.ones((1, O))), jnp.float32)

    const = lambda b: (0, 0)
    call = pl.pallas_call(
        _agg_kernel,
        grid=(B // R,),
        in_specs=[
            pl.BlockSpec((R, M + 8, D), lambda b: (b, 0, 0)),
            pl.BlockSpec((D, 2 * HO), const),
            pl.BlockSpec((1, 2 * HO), const),
            pl.BlockSpec((D, HO), const),
            pl.BlockSpec((1, HO), const),
            pl.BlockSpec((R, HO, SH), lambda b: (0, 0, 0)),
            pl.BlockSpec((H, HO), const),
        ],
        out_specs=pl.BlockSpec((R, 1, OUTW), lambda b: (b, 0, 0)),
        out_shape=jax.ShapeDtypeStruct((B, 1, OUTW), jnp.float32),
        scratch_shapes=[pltpu.VMEM((M, SH), jnp.float32)],
        compiler_params=pltpu.CompilerParams(
            dimension_semantics=("parallel",)),
    )
    out = call(xall, wqk_f, bqk_f, wv_f, bv_f, seg, exp)
    return out.reshape(B, OUTW)[:, : HO + 1]


# speedup vs baseline: 1.7263x; 1.5102x over previous
import jax
import jax.numpy as jnp
import numpy as np
from jax.experimental import pallas as pl
from jax.experimental.pallas import tpu as pltpu

B = 16
M = 2048
D = 256
H = 8
A = 64
O = 64
HO = H * O
TM = 512
NT = M // TM
R = 8
SH = R * H
OUTW = HO + 128


def _agg_kernel(xf_ref, wqk_ref, bqk_ref, wv_ref, bv_ref,
                seg_ref, exp_ref, out_ref, sc_ref):
    xrows = [xf_ref[r].reshape(M + 8, D) for r in range(R)]
    ens = [jnp.sum(xr[M:, :]) for xr in xrows]

    for t in range(NT):
        s = None
        for r in range(R):
            xt = xrows[r][t * TM:(t + 1) * TM, :].astype(jnp.bfloat16)
            qk2 = jnp.dot(xt, wqk_ref[...],
                          preferred_element_type=jnp.float32) + bqk_ref[...]
            q = qk2[:, :HO]
            k = qk2[:, HO:]
            d = jnp.dot((q * k).astype(jnp.bfloat16), seg_ref[r],
                        preferred_element_type=jnp.float32)
            s = d if s is None else s + d
        sc_ref[t * TM:(t + 1) * TM, :] = s

    en_vec = jnp.concatenate(
        [jnp.full((1, H), ens[r], jnp.float32) for r in range(R)], axis=1)
    rows = jax.lax.broadcasted_iota(jnp.int32, (M, SH), 0)
    pm = jnp.where(rows.astype(jnp.float32) < en_vec, 1.0, 0.0)
    z = sc_ref[...] * pm
    zmax = jnp.max(jnp.maximum(z, 0.0), axis=0, keepdims=True)
    ez = jnp.exp(z - zmax) * pm
    denom = jnp.sum(ez, axis=0, keepdims=True) + 1.0
    wts = ez / denom

    for r in range(R):
        w_r = wts[:, H * r:H * (r + 1)]
        xbar = jax.lax.dot_general(w_r, xrows[r][:M, :],
                                   (((0,), (0,)), ((), ())),
                                   preferred_element_type=jnp.float32)
        hsum = jnp.sum(w_r, axis=0, keepdims=True)
        vout = jnp.dot(xbar, wv_ref[...], preferred_element_type=jnp.float32)
        acc = (jnp.sum(vout * exp_ref[...], axis=0, keepdims=True)
               + jnp.dot(hsum, exp_ref[...], preferred_element_type=jnp.float32)
               * bv_ref[...])
        frac = jnp.full((1, OUTW - HO), ens[r] * (1.0 / M), jnp.float32)
        out_ref[r] = jnp.concatenate([acc, frac], axis=1)


def kernel(x, Ws, bs, Wq, bq, Wk, bk, Wv, bv):
    xall = x

    wqk_f = jnp.concatenate([Ws @ Wq, Ws @ Wk], axis=1).astype(jnp.bfloat16)
    bqk_f = jnp.concatenate([bs @ Wq + bq, bs @ Wk + bk]).reshape(1, 2 * HO)
    wv_f = Ws @ Wv
    bv_f = (bs @ Wv + bv).reshape(1, HO)

    seg_np = np.zeros((R, HO, SH), np.float32)
    for r in range(R):
        for h in range(H):
            seg_np[r, h * A:(h + 1) * A, H * r + h] = 1.0 / np.sqrt(A)
    seg = jnp.asarray(seg_np, jnp.bfloat16)
    exp = jnp.asarray(np.kron(np.eye(H), np.ones((1, O))), jnp.float32)

    const = lambda b: (0, 0)
    call = pl.pallas_call(
        _agg_kernel,
        grid=(B // R,),
        in_specs=[
            pl.BlockSpec((R, M * D + M), lambda b: (b, 0)),
            pl.BlockSpec((D, 2 * HO), const),
            pl.BlockSpec((1, 2 * HO), const),
            pl.BlockSpec((D, HO), const),
            pl.BlockSpec((1, HO), const),
            pl.BlockSpec((R, HO, SH), lambda b: (0, 0, 0)),
            pl.BlockSpec((H, HO), const),
        ],
        out_specs=pl.BlockSpec((R, 1, OUTW), lambda b: (b, 0, 0)),
        out_shape=jax.ShapeDtypeStruct((B, 1, OUTW), jnp.float32),
        scratch_shapes=[pltpu.VMEM((M, SH), jnp.float32)],
        compiler_params=pltpu.CompilerParams(
            dimension_semantics=("parallel",)),
    )
    out = call(xall, wqk_f, bqk_f, wv_f, bv_f, seg, exp)
    return out.reshape(B, OUTW)[:, : HO + 1]
